# baseline (speedup 1.0000x reference)
import functools

import jax
import jax.numpy as jnp
from jax import lax
from jax.experimental import pallas as pl
from jax.experimental.pallas import tpu as pltpu
from jax.experimental.pallas import tpu_sc as plsc

B = 16384
DIM = 64
N_TYPES = 8
N_ITEMS = B * N_TYPES
NUM_CONT = 16
TYPE_SIZE = 100000

NC = 2
NS = 16
NW = NC * NS
ITEMS_PER_W = N_ITEMS // NW
CHUNK = 1024
N_CHUNKS = ITEMS_PER_W // CHUNK
IDX_MINOR = 128
IDX_ROWS = CHUNK // IDX_MINOR


def _sc_gather(ids2d, table):
    mesh = plsc.VectorSubcoreMesh(core_axis_name="c", subcore_axis_name="s")

    @functools.partial(
        pl.kernel,
        out_type=jax.ShapeDtypeStruct((N_ITEMS, DIM), jnp.float32),
        mesh=mesh,
        scratch_types=[
            pltpu.VMEM((IDX_ROWS, IDX_MINOR), jnp.int32),
            pltpu.VMEM((CHUNK, DIM), jnp.float32),
            pltpu.SemaphoreType.DMA,
        ],
        compiler_params=pltpu.CompilerParams(use_tc_tiling_on_sc=False),
    )
    def k(ids_hbm, table_hbm, out_hbm, idx_v, rows_v, sem):
        wid = lax.axis_index("s") * NC + lax.axis_index("c")
        offs = (lax.iota(jnp.int32, 16) % N_TYPES) * TYPE_SIZE
        for c in range(N_CHUNKS):
            base = pl.multiple_of(wid * ITEMS_PER_W + c * CHUNK, CHUNK)
            pltpu.sync_copy(
                ids_hbm.at[pl.ds(pl.multiple_of(base // IDX_MINOR, IDX_ROWS), IDX_ROWS)],
                idx_v,
            )
            for i in range(IDX_ROWS):
                for s in range(IDX_MINOR // 16):
                    sl = (i, pl.ds(s * 16, 16))
                    idx_v[sl] = idx_v[sl] + offs
            descs = []
            for i in range(IDX_ROWS):
                descs.append(pltpu.async_copy(
                    table_hbm.at[idx_v.at[i]],
                    rows_v.at[pl.ds(i * IDX_MINOR, IDX_MINOR)],
                    sem,
                ))
            for d in descs:
                d.wait()
            pltpu.sync_copy(rows_v, out_hbm.at[pl.ds(base, CHUNK)])

    return k(ids2d, table)


def _tc_assemble(disc, ca, ct):
    bs = 512

    def body(disc_ref, ca_ref, ct_ref, out_ref):
        out_ref[:, 0:N_TYPES, :] = disc_ref[...]
        out_ref[:, N_TYPES:, :] = (
            ca_ref[...][:, :, None] * ct_ref[...][None, :, :]
        )

    return pl.pallas_call(
        body,
        grid=(B // bs,),
        in_specs=[
            pl.BlockSpec((bs, N_TYPES, DIM), lambda i: (i, 0, 0)),
            pl.BlockSpec((bs, NUM_CONT), lambda i: (i, 0)),
            pl.BlockSpec((NUM_CONT, DIM), lambda i: (0, 0)),
        ],
        out_specs=pl.BlockSpec((bs, N_TYPES + NUM_CONT, DIM), lambda i: (i, 0, 0)),
        out_shape=jax.ShapeDtypeStruct((B, N_TYPES + NUM_CONT, DIM), jnp.float32),
    )(disc, ca, ct)


def kernel(discrete_actions, continuous_actions, discrete_table, continuous_table):
    ids2d = discrete_actions.reshape(N_ITEMS // IDX_MINOR, IDX_MINOR)
    rows = _sc_gather(ids2d, discrete_table)
    return _tc_assemble(
        rows.reshape(B, N_TYPES, DIM), continuous_actions, continuous_table
    )


# speedup vs baseline: 1.0708x; 1.0708x over previous
import functools

import jax
import jax.numpy as jnp
from jax import lax
from jax.experimental import pallas as pl
from jax.experimental.pallas import tpu as pltpu
from jax.experimental.pallas import tpu_sc as plsc

B = 16384
DIM = 64
N_TYPES = 8
N_ITEMS = B * N_TYPES
NUM_CONT = 16
TYPE_SIZE = 100000

NC = 2
NS = 16
NW = NC * NS
ITEMS_PER_W = N_ITEMS // NW
CHUNK = 1024
N_CHUNKS = ITEMS_PER_W // CHUNK
IDX_MINOR = 128
IDX_ROWS = CHUNK // IDX_MINOR


def _sc_gather(ids2d, table):
    mesh = plsc.VectorSubcoreMesh(core_axis_name="c", subcore_axis_name="s")

    @functools.partial(
        pl.kernel,
        out_type=jax.ShapeDtypeStruct((N_ITEMS, 2 * DIM), jnp.float32),
        mesh=mesh,
        scratch_types=[
            pltpu.VMEM((IDX_ROWS, IDX_MINOR), jnp.int32),
            pltpu.VMEM((CHUNK, DIM), jnp.float32),
            pltpu.SemaphoreType.DMA,
        ],
        compiler_params=pltpu.CompilerParams(use_tc_tiling_on_sc=False),
    )
    def k(ids_hbm, table_hbm, out_hbm, idx_v, rows_v, sem):
        wid = lax.axis_index("s") * NC + lax.axis_index("c")
        offs = (lax.iota(jnp.int32, 16) % N_TYPES) * TYPE_SIZE
        for c in range(N_CHUNKS):
            base = pl.multiple_of(wid * ITEMS_PER_W + c * CHUNK, CHUNK)
            pltpu.sync_copy(
                ids_hbm.at[pl.ds(pl.multiple_of(base // IDX_MINOR, IDX_ROWS), IDX_ROWS)],
                idx_v,
            )
            for i in range(IDX_ROWS):
                for s in range(IDX_MINOR // 16):
                    sl = (i, pl.ds(s * 16, 16))
                    idx_v[sl] = idx_v[sl] + offs
            descs = []
            for i in range(IDX_ROWS):
                descs.append(pltpu.async_copy(
                    table_hbm.at[idx_v.at[i]],
                    rows_v.at[pl.ds(i * IDX_MINOR, IDX_MINOR)],
                    sem,
                ))
            for d in descs:
                d.wait()
            pltpu.sync_copy(rows_v, out_hbm.at[pl.ds(base, CHUNK), pl.ds(0, DIM)])

    return k(ids2d, table)


def _tc_assemble(disc, ca, ct):
    bs = 512

    def body(disc_ref, ca_ref, ct_ref, out_ref):
        out_ref[:, 0:N_TYPES, :] = disc_ref[:, 0:DIM].reshape(bs, N_TYPES, DIM)
        out_ref[:, N_TYPES:, :] = (
            ca_ref[...][:, :, None] * ct_ref[...][None, :, :]
        )

    return pl.pallas_call(
        body,
        grid=(B // bs,),
        in_specs=[
            pl.BlockSpec((bs * N_TYPES, 2 * DIM), lambda i: (i, 0)),
            pl.BlockSpec((bs, NUM_CONT), lambda i: (i, 0)),
            pl.BlockSpec((NUM_CONT, DIM), lambda i: (0, 0)),
        ],
        out_specs=pl.BlockSpec((bs, N_TYPES + NUM_CONT, DIM), lambda i: (i, 0, 0)),
        out_shape=jax.ShapeDtypeStruct((B, N_TYPES + NUM_CONT, DIM), jnp.float32),
    )(disc, ca, ct)


def kernel(discrete_actions, continuous_actions, discrete_table, continuous_table):
    ids2d = discrete_actions.reshape(N_ITEMS // IDX_MINOR, IDX_MINOR)
    rows = _sc_gather(ids2d, discrete_table)
    return _tc_assemble(rows, continuous_actions, continuous_table)
